# baseline (speedup 1.0000x reference)
import functools

import jax
import jax.numpy as jnp
from jax import lax
from jax.experimental import pallas as pl
from jax.experimental.pallas import tpu as pltpu
from jax.experimental.pallas import tpu_sc as plsc

DIM = 16
VOCAB_PAD = 48


def _lut_body(table_ref, pos_ref, gamma_ref, beta_ref, out_ref):
    emb = table_ref[...] + pos_ref[...]
    mean = jnp.mean(emb, axis=-1, keepdims=True)
    var = jnp.mean((emb - mean) * (emb - mean), axis=-1, keepdims=True)
    normed = (emb - mean) * lax.rsqrt(var + 1e-5)
    out_ref[...] = jax.nn.sigmoid(normed * gamma_ref[...] + beta_ref[...])


def _compute_lut(emb_table, gamma, beta):
    v = emb_table.shape[0]
    table_p = jnp.pad(emb_table, ((0, VOCAB_PAD - v), (0, 0)))
    pos = jnp.arange(DIM, dtype=jnp.float32).reshape(1, DIM)
    return pl.pallas_call(
        _lut_body,
        out_shape=jax.ShapeDtypeStruct((VOCAB_PAD, DIM), jnp.float32),
    )(table_p, pos, gamma.reshape(1, DIM), beta.reshape(1, DIM))


NC, NS = 2, 16
NW = NC * NS
IW = 512
JC = 40
JSTRIDE = 41
LSTRIDE = 17
TILE = 1024
CTBLK = 4 * TILE


def _make_gather(B, L):
    n_jg = L // JC
    mesh = plsc.VectorSubcoreMesh(core_axis_name="c", subcore_axis_name="s")

    @functools.partial(
        pl.kernel,
        mesh=mesh,
        out_type=jax.ShapeDtypeStruct((B * L * DIM,), jnp.float32),
        scratch_types=[
            pltpu.VMEM((VOCAB_PAD * DIM,), jnp.float32),
            pltpu.VMEM((VOCAB_PAD * LSTRIDE,), jnp.float32),
            pltpu.VMEM((IW, JSTRIDE), jnp.int32),
            pltpu.VMEM((IW, JSTRIDE), jnp.int32),
            pltpu.VMEM((2 * CTBLK,), jnp.float32),
            pltpu.VMEM((2 * CTBLK,), jnp.float32),
        ]
        + [pltpu.SemaphoreType.DMA] * 4,
        compiler_params=pltpu.CompilerParams(
            use_tc_tiling_on_sc=False, needs_layout_passes=False
        ),
    )
    def gather(lut_hbm, x_hbm, out_hbm, lut_v, lut17, idx0, idx1, ob0, ob1, *sems):
        si = sems[0:2]
        so = sems[2:4]
        idx_bufs = (idx0, idx1)
        out_bufs = (ob0, ob1)
        wid = lax.axis_index("s") * NC + lax.axis_index("c")
        i0 = wid * IW
        obase = i0 * 8
        iota = lax.iota(jnp.int32, 16)
        pltpu.sync_copy(lut_hbm, lut_v)
        for v in range(VOCAB_PAD):
            plsc.store_scatter(
                lut17, [iota + v * LSTRIDE], lut_v[pl.ds(v * DIM, 16)]
            )

        def idx_desc(jg, b):
            return pltpu.make_async_copy(
                x_hbm.at[pl.ds(i0, IW), pl.ds(jg * JC, JC)],
                idx_bufs[b].at[:, pl.ds(0, JC)],
                si[b],
            )

        jblk = B * DIM
        ctoff = B * 8

        def out_desc(j, ct, b):
            return pltpu.make_async_copy(
                out_bufs[b].at[pl.ds(ct * CTBLK, CTBLK)],
                out_hbm.at[pl.ds(j * jblk + ct * ctoff + obase, CTBLK)],
                so[b],
            )

        def compute_j(jc, j, bb, idx_b, first):
            ob = out_bufs[bb]

            def _drain():
                out_desc(j - 2, 0, bb).wait()
                out_desc(j - 2, 1, bb).wait()

            if not first:
                _drain()

            colj = jnp.full((16,), 0, jnp.int32) + jc

            @plsc.parallel_loop(0, IW, 16, unroll=4)
            def grp(r0):
                rows = iota + r0
                a17 = plsc.load_gather(idx_b, [rows, colj]) * LSTRIDE
                g16 = r0 // 16
                off = (g16 // 8) * TILE + (g16 % 8) * 16
                for c in range(DIM):
                    s = (c // 8) * CTBLK + (c % 8) * 128
                    v = plsc.load_gather(lut17, [a17 + c])
                    ob[pl.ds(off + s, 16)] = v

            out_desc(j, 0, bb).start()
            out_desc(j, 1, bb).start()

        idx_desc(0, 0).start()
        for jg in range(n_jg):
            bi = jg % 2
            idx_desc(jg, bi).wait()
            if jg + 1 < n_jg:
                idx_desc(jg + 1, 1 - bi).start()
            idx_b = idx_bufs[bi]

            def jcpair(p, carry):
                for b in range(2):
                    jc = p * 2 + b
                    compute_j(jc, jg * JC + jc, b, idx_b, first=False)
                return carry

            if jg == 0:
                for b in range(2):
                    compute_j(b, b, b, idx_b, first=True)
                lax.fori_loop(1, JC // 2, jcpair, 0)
            else:
                lax.fori_loop(0, JC // 2, jcpair, 0)
        for b in range(2):
            j = L - 2 + b
            out_desc(j, 0, b).wait()
            out_desc(j, 1, b).wait()

    return gather


def kernel(x, emb_table, gamma, beta):
    b, l = x.shape
    lut = _compute_lut(emb_table, gamma, beta)
    flat = _make_gather(b, l)(lut.reshape(-1), x.astype(jnp.int32))
    p5 = flat.reshape(l, 2, b // 128, 8, 128)
    return p5.transpose(2, 4, 0, 1, 3).reshape(b, l, DIM)


# speedup vs baseline: 1.0176x; 1.0176x over previous
import functools

import jax
import jax.numpy as jnp
from jax import lax
from jax.experimental import pallas as pl
from jax.experimental.pallas import tpu as pltpu
from jax.experimental.pallas import tpu_sc as plsc

DIM = 16
VOCAB_PAD = 48


def _lut_body(table_ref, pos_ref, gamma_ref, beta_ref, out_ref):
    emb = table_ref[...] + pos_ref[...]
    mean = jnp.mean(emb, axis=-1, keepdims=True)
    var = jnp.mean((emb - mean) * (emb - mean), axis=-1, keepdims=True)
    normed = (emb - mean) * lax.rsqrt(var + 1e-5)
    out_ref[...] = jax.nn.sigmoid(normed * gamma_ref[...] + beta_ref[...])


def _compute_lut(emb_table, gamma, beta):
    v = emb_table.shape[0]
    table_p = jnp.pad(emb_table, ((0, VOCAB_PAD - v), (0, 0)))
    pos = jnp.arange(DIM, dtype=jnp.float32).reshape(1, DIM)
    return pl.pallas_call(
        _lut_body,
        out_shape=jax.ShapeDtypeStruct((VOCAB_PAD, DIM), jnp.float32),
    )(table_p, pos, gamma.reshape(1, DIM), beta.reshape(1, DIM))


NC, NS = 2, 16
NW = NC * NS
IW = 512
JC = 40
JSTRIDE = 41
LSTRIDE = 17
TILE = 1024
CTBLK = 4 * TILE


def _make_gather(B, L):
    n_jg = L // JC
    mesh = plsc.VectorSubcoreMesh(core_axis_name="c", subcore_axis_name="s")

    @functools.partial(
        pl.kernel,
        mesh=mesh,
        out_type=jax.ShapeDtypeStruct((B * L * DIM,), jnp.float32),
        scratch_types=[
            pltpu.VMEM((VOCAB_PAD * DIM,), jnp.float32),
            pltpu.VMEM((VOCAB_PAD * LSTRIDE,), jnp.float32),
            pltpu.VMEM((IW, JSTRIDE), jnp.int32),
            pltpu.VMEM((IW, JSTRIDE), jnp.int32),
            pltpu.VMEM((2 * CTBLK,), jnp.float32),
            pltpu.VMEM((2 * CTBLK,), jnp.float32),
        ]
        + [pltpu.SemaphoreType.DMA] * 4,
        compiler_params=pltpu.CompilerParams(
            use_tc_tiling_on_sc=False, needs_layout_passes=False
        ),
    )
    def gather(lut_hbm, x_hbm, out_hbm, lut_v, lut17, idx0, idx1, ob0, ob1, *sems):
        si = sems[0:2]
        so = sems[2:4]
        idx_bufs = (idx0, idx1)
        out_bufs = (ob0, ob1)
        wid = lax.axis_index("s") * NC + lax.axis_index("c")
        i0 = wid * IW
        obase = i0 * 8
        iota = lax.iota(jnp.int32, 16)
        pltpu.sync_copy(lut_hbm, lut_v)
        for v in range(VOCAB_PAD):
            plsc.store_scatter(
                lut17, [iota + v * LSTRIDE], lut_v[pl.ds(v * DIM, 16)]
            )

        def idx_desc(jg, b):
            return pltpu.make_async_copy(
                x_hbm.at[pl.ds(i0, IW), pl.ds(jg * JC, JC)],
                idx_bufs[b].at[:, pl.ds(0, JC)],
                si[b],
            )

        jblk = B * DIM
        ctoff = B * 8

        def out_desc(j, ct, b):
            return pltpu.make_async_copy(
                out_bufs[b].at[pl.ds(ct * CTBLK, CTBLK)],
                out_hbm.at[pl.ds(j * jblk + ct * ctoff + obase, CTBLK)],
                so[b],
            )

        def compute_j(jc, j, bb, idx_b, first):
            ob = out_bufs[bb]

            def _drain():
                out_desc(j - 2, 0, bb).wait()
                out_desc(j - 2, 1, bb).wait()

            if not first:
                _drain()

            colj = jnp.full((16,), 0, jnp.int32) + jc

            @plsc.parallel_loop(0, IW, 16, unroll=2)
            def grp(r0):
                rows = iota + r0
                a17 = plsc.load_gather(idx_b, [rows, colj]) * LSTRIDE
                g16 = r0 // 16
                off = (g16 // 8) * TILE + (g16 % 8) * 16
                for c in range(DIM):
                    s = (c // 8) * CTBLK + (c % 8) * 128
                    v = plsc.load_gather(lut17, [a17 + c])
                    ob[pl.ds(off + s, 16)] = v

            out_desc(j, 0, bb).start()
            out_desc(j, 1, bb).start()

        idx_desc(0, 0).start()
        for jg in range(n_jg):
            bi = jg % 2
            idx_desc(jg, bi).wait()
            if jg + 1 < n_jg:
                idx_desc(jg + 1, 1 - bi).start()
            idx_b = idx_bufs[bi]

            def jcpair(p, carry):
                for b in range(2):
                    jc = p * 2 + b
                    compute_j(jc, jg * JC + jc, b, idx_b, first=False)
                return carry

            if jg == 0:
                for b in range(2):
                    compute_j(b, b, b, idx_b, first=True)
                lax.fori_loop(1, JC // 2, jcpair, 0)
            else:
                lax.fori_loop(0, JC // 2, jcpair, 0)
        for b in range(2):
            j = L - 2 + b
            out_desc(j, 0, b).wait()
            out_desc(j, 1, b).wait()

    return gather


def kernel(x, emb_table, gamma, beta):
    b, l = x.shape
    lut = _compute_lut(emb_table, gamma, beta)
    flat = _make_gather(b, l)(lut.reshape(-1), x.astype(jnp.int32))
    p5 = flat.reshape(l, 2, b // 128, 8, 128)
    return p5.transpose(2, 4, 0, 1, 3).reshape(b, l, DIM)
